# baseline (speedup 1.0000x reference)
import functools

import jax
import jax.numpy as jnp
from jax import lax
from jax.experimental import pallas as pl
from jax.experimental.pallas import tpu as pltpu
from jax.experimental.pallas import tpu_sc as plsc

N = 10000
E = 320000
D = 128
G = 64
C = 10

NC = 2
NS = 16
NW = NC * NS
K = 128
CH = 80
E_PAD = NW * K * CH
N_PAD = 10240
RPT = N_PAD // NS

@functools.cache
def _get_mesh():
    return plsc.VectorSubcoreMesh(
        core_axis_name="c", subcore_axis_name="s", num_cores=NC, num_subcores=NS
    )


def _sc_degree_body(eidx_hbm, z_hbm, degp_hbm, eidx_v, hist_v):
    c = lax.axis_index("c")
    s = lax.axis_index("s")
    wid = s * NC + c
    pltpu.sync_copy(z_hbm, hist_v)
    pltpu.sync_copy(eidx_hbm.at[wid], eidx_v)
    ones = jnp.ones((16,), jnp.float32)

    @pl.loop(0, CH)
    def _(j):
        @pl.loop(0, K // 16)
        def _(i):
            idx = eidx_v[j, 1, pl.ds(i * 16, 16)]
            plsc.addupdate_scatter(hist_v, [idx], ones)

    pltpu.sync_copy(hist_v, degp_hbm.at[wid])


IB = 40
NBUF = 2


def _sc_aggregate_body(s_hbm, eidx_hbm, z_hbm, acc_hbm,
                       eidx_v, b0, b1, acc_sh,
                       sem0, sem1):
    c = lax.axis_index("c")
    s = lax.axis_index("s")
    wid = s * NC + c
    r0 = s * RPT
    bufs = (b0, b1)
    sems = (sem0, sem1)
    pltpu.sync_copy(z_hbm.at[pl.ds(r0, RPT)], acc_sh.at[pl.ds(r0, RPT)])
    plsc.subcore_barrier()

    @pl.loop(0, CH, step=IB)
    def _(j0):
        pltpu.sync_copy(eidx_hbm.at[wid, pl.ds(j0, IB)], eidx_v)
        descs = {}
        descs[0] = pltpu.async_copy(
            s_hbm.at[eidx_v.at[0, 0]], bufs[0], sems[0])
        for t in range(IB):
            if t + 1 < IB:
                nb = (t + 1) % NBUF
                descs[t + 1] = pltpu.async_copy(
                    s_hbm.at[eidx_v.at[t + 1, 0]], bufs[nb], sems[nb])
            descs[t].wait()
            pltpu.sync_copy(bufs[t % NBUF], acc_sh.at[eidx_v.at[t, 1]],
                            add=True)

    plsc.subcore_barrier()
    pltpu.sync_copy(acc_sh.at[pl.ds(r0, RPT)], acc_hbm.at[c, pl.ds(r0, RPT)])


@functools.cache
def _sc_kernels():
    mesh = _get_mesh()
    sc_degree = pl.kernel(
        _sc_degree_body,
        out_type=jax.ShapeDtypeStruct((NW, N_PAD), jnp.float32),
        mesh=mesh,
        scratch_types=[
            pltpu.VMEM((CH, 2, K), jnp.int32),
            pltpu.VMEM((N_PAD,), jnp.float32),
        ],
        compiler_params=pltpu.CompilerParams(needs_layout_passes=False),
    )
    sc_aggregate = pl.kernel(
        _sc_aggregate_body,
        out_type=jax.ShapeDtypeStruct((NC, N_PAD, D), jnp.float32),
        mesh=mesh,
        scratch_types=[
            pltpu.VMEM((IB, 2, K), jnp.int32),
            pltpu.VMEM((K, D), jnp.float32),
            pltpu.VMEM((K, D), jnp.float32),
            pltpu.VMEM_SHARED((N_PAD, D), jnp.float32),
            pltpu.SemaphoreType.DMA,
            pltpu.SemaphoreType.DMA,
        ],
    )
    return sc_degree, sc_aggregate


def _tc_matmul(x_ref, w_ref, h_ref):
    h_ref[...] = jnp.dot(x_ref[...], w_ref[...],
                         preferred_element_type=jnp.float32)


def _tc_scale(h_ref, degp_ref, dis_ref, s_ref):
    deg = 1.0 + lax.dot_general(
        degp_ref[...], jnp.ones((NW, 1), jnp.float32),
        (((0,), (0,)), ((), ())), preferred_element_type=jnp.float32)
    row = lax.broadcasted_iota(jnp.int32, (N_PAD, 1), 0)
    dis = jnp.where(row < N, lax.rsqrt(deg), 0.0)
    dis_ref[...] = dis
    s_ref[...] = h_ref[...] * dis


def _tc_mid(acc_ref, s_ref, dis_ref, b_ref, w_ref, out_ref):
    dis = dis_ref[...]
    h = dis * (acc_ref[0] + acc_ref[1] + s_ref[...]) + b_ref[...]
    h = jnp.maximum(h, 0.0)
    out_ref[...] = (
        jnp.dot(h, w_ref[...], preferred_element_type=jnp.float32) * dis
    )


def _tc_final(acc_ref, s_ref, dis_ref, b_ref, batch_ref, wl_ref, bl_ref,
              out_ref):
    h = dis_ref[...] * (acc_ref[0] + acc_ref[1] + s_ref[...]) + b_ref[...]
    gid = lax.broadcasted_iota(jnp.int32, (G, N_PAD), 0)
    oh = (batch_ref[...] == gid).astype(jnp.float32)
    sums = jnp.dot(oh, h, preferred_element_type=jnp.float32)
    cnt = jnp.sum(oh, axis=1)[:, None]
    g = sums / jnp.maximum(cnt, 1.0)
    out_ref[...] = (
        jnp.dot(g, wl_ref[...], preferred_element_type=jnp.float32)
        + bl_ref[...]
    )


def kernel(x, edge_index, batch, W1, b1, W2, b2, W3, b3, Wl, bl):
    x = x.astype(jnp.float32)
    ei = edge_index.astype(jnp.int32)
    pad_e = E_PAD - E
    pad_src = jnp.arange(pad_e, dtype=jnp.int32) % N_PAD
    pad_dst = N + (jnp.arange(pad_e, dtype=jnp.int32) % (N_PAD - N))
    src_p = jnp.concatenate([ei[0], pad_src]).reshape(NW, CH, K)
    dst_p = jnp.concatenate([ei[1], pad_dst]).reshape(NW, CH, K)
    eidx = jnp.stack([src_p, dst_p], axis=2)
    xp = jnp.concatenate([x, jnp.zeros((N_PAD - N, D), jnp.float32)])
    batch_row = jnp.concatenate(
        [batch.astype(jnp.int32), jnp.full((N_PAD - N,), G, jnp.int32)]
    ).reshape(1, N_PAD)
    z128 = jnp.zeros((N_PAD, D), jnp.float32)
    z1d = jnp.zeros((N_PAD,), jnp.float32)

    _sc_degree, _sc_aggregate = _sc_kernels()
    degp = _sc_degree(eidx, z1d)

    h1 = pl.pallas_call(
        _tc_matmul,
        out_shape=jax.ShapeDtypeStruct((N_PAD, D), jnp.float32),
    )(xp, W1)
    dis, s1 = pl.pallas_call(
        _tc_scale,
        out_shape=(
            jax.ShapeDtypeStruct((N_PAD, 1), jnp.float32),
            jax.ShapeDtypeStruct((N_PAD, D), jnp.float32),
        ),
    )(h1, degp)

    mid = pl.pallas_call(
        _tc_mid,
        out_shape=jax.ShapeDtypeStruct((N_PAD, D), jnp.float32),
    )

    acc1 = _sc_aggregate(s1, eidx, z128)
    s2 = mid(acc1, s1, dis, b1.reshape(1, D), W2)
    acc2 = _sc_aggregate(s2, eidx, z128)
    s3 = mid(acc2, s2, dis, b2.reshape(1, D), W3)
    acc3 = _sc_aggregate(s3, eidx, z128)

    out = pl.pallas_call(
        _tc_final,
        out_shape=jax.ShapeDtypeStruct((G, C), jnp.float32),
    )(acc3, s3, dis, b3.reshape(1, D), batch_row, Wl, bl.reshape(1, C))
    return out


# speedup vs baseline: 1.0020x; 1.0020x over previous
import functools

import jax
import jax.numpy as jnp
from jax import lax
from jax.experimental import pallas as pl
from jax.experimental.pallas import tpu as pltpu
from jax.experimental.pallas import tpu_sc as plsc

N = 10000
E = 320000
D = 128
G = 64
C = 10

NC = 2
NS = 16
NW = NC * NS
K = 128
CH = 80
E_PAD = NW * K * CH
N_PAD = 10240
RPT = N_PAD // NS

@functools.cache
def _get_mesh():
    return plsc.VectorSubcoreMesh(
        core_axis_name="c", subcore_axis_name="s", num_cores=NC, num_subcores=NS
    )


def _sc_degree_body(eidx_hbm, z_hbm, degp_hbm, eidx_v, hist_v):
    c = lax.axis_index("c")
    s = lax.axis_index("s")
    wid = s * NC + c
    pltpu.sync_copy(z_hbm, hist_v)
    pltpu.sync_copy(eidx_hbm.at[wid], eidx_v)
    ones = jnp.ones((16,), jnp.float32)

    @pl.loop(0, CH)
    def _(j):
        @pl.loop(0, K // 16)
        def _(i):
            idx = eidx_v[j, 1, pl.ds(i * 16, 16)]
            plsc.addupdate_scatter(hist_v, [idx], ones)

    pltpu.sync_copy(hist_v, degp_hbm.at[wid])


IB = 40
NBUF = 2


def _sc_aggregate_body(s_hbm, eidx_hbm, z_hbm, acc_hbm,
                       eidx_v, b0, b1, acc_sh,
                       sem0, sem1):
    c = lax.axis_index("c")
    s = lax.axis_index("s")
    wid = s * NC + c
    r0 = s * RPT
    bufs = (b0, b1)
    sems = (sem0, sem1)
    pltpu.sync_copy(z_hbm.at[pl.ds(r0, RPT)], acc_sh.at[pl.ds(r0, RPT)])
    plsc.subcore_barrier()

    @pl.loop(0, CH, step=IB)
    def _(j0):
        pltpu.sync_copy(eidx_hbm.at[wid, pl.ds(j0, IB)], eidx_v)
        descs = {}
        descs[0] = pltpu.async_copy(
            s_hbm.at[eidx_v.at[0, 0]], bufs[0], sems[0])
        for t in range(IB):
            if t + 1 < IB:
                nb = (t + 1) % NBUF
                descs[t + 1] = pltpu.async_copy(
                    s_hbm.at[eidx_v.at[t + 1, 0]], bufs[nb], sems[nb])
            descs[t].wait()
            pltpu.sync_copy(bufs[t % NBUF], acc_sh.at[eidx_v.at[t, 1]],
                            add=True)

    plsc.subcore_barrier()
    pltpu.sync_copy(acc_sh.at[pl.ds(r0, RPT)], acc_hbm.at[c, pl.ds(r0, RPT)])


@functools.cache
def _sc_kernels():
    mesh = _get_mesh()
    sc_degree = pl.kernel(
        _sc_degree_body,
        out_type=jax.ShapeDtypeStruct((NW, N_PAD), jnp.float32),
        mesh=mesh,
        scratch_types=[
            pltpu.VMEM((CH, 2, K), jnp.int32),
            pltpu.VMEM((N_PAD,), jnp.float32),
        ],
        compiler_params=pltpu.CompilerParams(needs_layout_passes=False),
    )
    sc_aggregate = pl.kernel(
        _sc_aggregate_body,
        out_type=jax.ShapeDtypeStruct((NC, N_PAD, D), jnp.float32),
        mesh=mesh,
        scratch_types=[
            pltpu.VMEM((IB, 2, K), jnp.int32),
            pltpu.VMEM((K, D), jnp.float32),
            pltpu.VMEM((K, D), jnp.float32),
            pltpu.VMEM_SHARED((N_PAD, D), jnp.float32),
            pltpu.SemaphoreType.DMA,
            pltpu.SemaphoreType.DMA,
        ],
    )
    return sc_degree, sc_aggregate


def _tc_first(x_ref, w_ref, degp_ref, dis_ref, s_ref):
    deg = 1.0 + lax.dot_general(
        degp_ref[...], jnp.ones((NW, 1), jnp.float32),
        (((0,), (0,)), ((), ())), preferred_element_type=jnp.float32)
    row = lax.broadcasted_iota(jnp.int32, (N_PAD, 1), 0)
    dis = jnp.where(row < N, lax.rsqrt(deg), 0.0)
    dis_ref[...] = dis
    h = jnp.dot(x_ref[...], w_ref[...], preferred_element_type=jnp.float32)
    s_ref[...] = h * dis


def _tc_mid(acc_ref, s_ref, dis_ref, b_ref, w_ref, out_ref):
    dis = dis_ref[...]
    h = dis * (acc_ref[0] + acc_ref[1] + s_ref[...]) + b_ref[...]
    h = jnp.maximum(h, 0.0)
    out_ref[...] = (
        jnp.dot(h, w_ref[...], preferred_element_type=jnp.float32) * dis
    )


def _tc_final(acc_ref, s_ref, dis_ref, b_ref, batch_ref, wl_ref, bl_ref,
              out_ref):
    h = dis_ref[...] * (acc_ref[0] + acc_ref[1] + s_ref[...]) + b_ref[...]
    gid = lax.broadcasted_iota(jnp.int32, (G, N_PAD), 0)
    oh = (batch_ref[...] == gid).astype(jnp.float32)
    sums = jnp.dot(oh, h, preferred_element_type=jnp.float32)
    cnt = jnp.sum(oh, axis=1)[:, None]
    g = sums / jnp.maximum(cnt, 1.0)
    out_ref[...] = (
        jnp.dot(g, wl_ref[...], preferred_element_type=jnp.float32)
        + bl_ref[...]
    )


def kernel(x, edge_index, batch, W1, b1, W2, b2, W3, b3, Wl, bl):
    x = x.astype(jnp.float32)
    ei = edge_index.astype(jnp.int32)
    pad_e = E_PAD - E
    pad_src = jnp.arange(pad_e, dtype=jnp.int32) % N_PAD
    pad_dst = N + (jnp.arange(pad_e, dtype=jnp.int32) % (N_PAD - N))
    src_p = jnp.concatenate([ei[0], pad_src]).reshape(NW, CH, K)
    dst_p = jnp.concatenate([ei[1], pad_dst]).reshape(NW, CH, K)
    eidx = jnp.stack([src_p, dst_p], axis=2)
    xp = jnp.concatenate([x, jnp.zeros((N_PAD - N, D), jnp.float32)])
    batch_row = jnp.concatenate(
        [batch.astype(jnp.int32), jnp.full((N_PAD - N,), G, jnp.int32)]
    ).reshape(1, N_PAD)
    z128 = jnp.zeros((N_PAD, D), jnp.float32)
    z1d = jnp.zeros((N_PAD,), jnp.float32)

    _sc_degree, _sc_aggregate = _sc_kernels()
    degp = _sc_degree(eidx, z1d)

    dis, s1 = pl.pallas_call(
        _tc_first,
        out_shape=(
            jax.ShapeDtypeStruct((N_PAD, 1), jnp.float32),
            jax.ShapeDtypeStruct((N_PAD, D), jnp.float32),
        ),
    )(xp, W1, degp)

    mid = pl.pallas_call(
        _tc_mid,
        out_shape=jax.ShapeDtypeStruct((N_PAD, D), jnp.float32),
    )

    acc1 = _sc_aggregate(s1, eidx, z128)
    s2 = mid(acc1, s1, dis, b1.reshape(1, D), W2)
    acc2 = _sc_aggregate(s2, eidx, z128)
    s3 = mid(acc2, s2, dis, b2.reshape(1, D), W3)
    acc3 = _sc_aggregate(s3, eidx, z128)

    out = pl.pallas_call(
        _tc_final,
        out_shape=jax.ShapeDtypeStruct((G, C), jnp.float32),
    )(acc3, s3, dis, b3.reshape(1, D), batch_row, Wl, bl.reshape(1, C))
    return out
